# Initial kernel scaffold; baseline (speedup 1.0000x reference)
import jax
import jax.numpy as jnp
from jax.experimental import pallas as pl


def kernel(coords, densities, grid):
    raise NotImplementedError("write your pallas kernel here")


# speedup vs baseline: 1.7919x; 1.7919x over previous
import jax
import jax.numpy as jnp
from jax import lax
from jax.experimental import pallas as pl
from jax.experimental.pallas import tpu as pltpu
from jax.experimental.pallas import tpu_sc as plsc

RES = 128
THRESH = 0.01
N = 2097152
N_CELLS = RES * RES * RES
PAD = 8192
TOT = N_CELLS + PAD

NUM_TILES = 16
NPT = N // NUM_TILES
CHUNK = 8192
ROWS = CHUNK // 128
NCHUNK = NPT // CHUNK


def _body(coords_ref, dens_ref, grid_ref, out_ref, xv, yv, zv, dv, idxbuf,
          ones):
    sid = lax.axis_index("s")

    for i in range(8):
        ones[pl.ds(i * 16, 16)] = jnp.full((16,), 1.0, jnp.float32)

    slab = N_CELLS // NUM_TILES
    pltpu.sync_copy(grid_ref.at[pl.ds(sid * slab, slab)],
                    out_ref.at[pl.ds(sid * slab, slab)])
    padslab = PAD // NUM_TILES
    pltpu.sync_copy(grid_ref.at[pl.ds(sid * padslab, padslab)],
                    out_ref.at[pl.ds(N_CELLS + sid * padslab, padslab)])
    plsc.subcore_barrier()

    lane = lax.iota(jnp.int32, 16)

    @pl.loop(0, NCHUNK)
    def _chunk(k):
        base = sid * NPT + k * CHUNK
        pltpu.sync_copy(coords_ref.at[pl.ds(base, CHUNK)], xv)
        pltpu.sync_copy(coords_ref.at[pl.ds(N + base, CHUNK)], yv)
        pltpu.sync_copy(coords_ref.at[pl.ds(2 * N + base, CHUNK)], zv)
        pltpu.sync_copy(dens_ref.at[pl.ds(base, CHUNK)], dv)

        @pl.loop(0, ROWS)
        def _row(r):
            for g in range(8):
                pid = r * 128 + g * 16 + lane
                off = r * 128 + g * 16
                x = xv[pl.ds(off, 16)]
                y = yv[pl.ds(off, 16)]
                z = zv[pl.ds(off, 16)]
                ix = jnp.clip((x * 127.0).astype(jnp.int32), 0, RES - 1)
                iy = jnp.clip((y * 127.0).astype(jnp.int32), 0, RES - 1)
                iz = jnp.clip((z * 127.0).astype(jnp.int32), 0, RES - 1)
                lin = (ix * RES + iy) * RES + iz
                d = dv[pl.ds(off, 16)]
                trash = N_CELLS + (pid & (PAD - 1))
                idxbuf[r, pl.ds(g * 16, 16)] = jnp.where(d > THRESH, lin, trash)

        @pl.loop(0, ROWS)
        def _scat(r):
            pltpu.sync_copy(ones, out_ref.at[idxbuf.at[r]])


_mesh = plsc.VectorSubcoreMesh(
    core_axis_name="c", subcore_axis_name="s", num_cores=1)

_scatter = pl.kernel(
    _body,
    out_type=jax.ShapeDtypeStruct((TOT,), jnp.float32),
    mesh=_mesh,
    scratch_types=[
        pltpu.VMEM((CHUNK,), jnp.float32),
        pltpu.VMEM((CHUNK,), jnp.float32),
        pltpu.VMEM((CHUNK,), jnp.float32),
        pltpu.VMEM((CHUNK,), jnp.float32),
        pltpu.VMEM((ROWS, 128), jnp.int32),
        pltpu.VMEM((128,), jnp.float32),
    ],
)


@jax.jit
def kernel(coords, densities, grid):
    coords_t = coords.T.reshape(-1)
    out = _scatter(coords_t, densities, grid.reshape(-1))
    return out[:N_CELLS].reshape(RES, RES, RES)
